# Initial kernel scaffold; baseline (speedup 1.0000x reference)
import jax
import jax.numpy as jnp
from jax.experimental import pallas as pl


def kernel(x, edge_index, batch, params):
    raise NotImplementedError("write your pallas kernel here")


# speedup vs baseline: 1.5417x; 1.5417x over previous
import functools
from math import ceil

import jax
import jax.numpy as jnp
from jax import lax
from jax.experimental import pallas as pl
from jax.experimental.pallas import tpu as pltpu
from jax.experimental.pallas import tpu_sc as plsc

N_REAL = 10000
NPAD = 10240
E_REAL = 160000
EPAD = 163840
NSUB = 16
EPW = EPAD // NSUB
EBLK = 128
NBLK = EPW // EBLK
ROWS_PW = NPAD // NSUB
CCHUNK = 128
TILE = 1024
DUMMY_DST = N_REAL + 8


@functools.cache
def _make_segsum_sc():
    @functools.partial(
        pl.kernel,
        mesh=plsc.VectorSubcoreMesh(
            core_axis_name="c", subcore_axis_name="s", num_cores=1),
        out_type=jax.ShapeDtypeStruct((NPAD, CCHUNK), jnp.float32),
        scratch_types=[
            pltpu.VMEM((EBLK,), jnp.int32),
            pltpu.VMEM((EBLK,), jnp.int32),
            pltpu.VMEM((EBLK, CCHUNK), jnp.float32),
            pltpu.VMEM_SHARED((NPAD, CCHUNK), jnp.float32),
            pltpu.SemaphoreType.DMA,
        ],
    )
    def _segsum_sc(h_hbm, src_hbm, dst_hbm, zeros_hbm, out_hbm,
                   sidx, didx, rows, acc, sem):
        wid = lax.axis_index("s")
        base = wid * ROWS_PW
        pltpu.sync_copy(zeros_hbm.at[pl.ds(base, ROWS_PW)],
                        acc.at[pl.ds(base, ROWS_PW)])
        plsc.subcore_barrier()

        def body(b, carry):
            pltpu.sync_copy(src_hbm.at[wid, b], sidx)
            pltpu.sync_copy(dst_hbm.at[wid, b], didx)
            pltpu.async_copy(h_hbm.at[sidx], rows, sem).wait()
            pltpu.sync_copy(rows, acc.at[didx], add=True)
            return carry

        lax.fori_loop(0, NBLK, body, 0)
        plsc.subcore_barrier()
        pltpu.sync_copy(acc.at[pl.ds(base, ROWS_PW)],
                        out_hbm.at[pl.ds(base, ROWS_PW)])

    return _segsum_sc


def _segment_sum(h, src3, dst3, zeros_chunk):
    d = h.shape[1]
    fn = _make_segsum_sc()
    outs = []
    for c in range(d // CCHUNK):
        hc = lax.slice_in_dim(h, c * CCHUNK, (c + 1) * CCHUNK, axis=1)
        outs.append(fn(hc, src3, dst3, zeros_chunk))
    return jnp.concatenate(outs, axis=1)


def _row_valid(i, rows):
    gid = i * TILE + lax.broadcasted_iota(jnp.int32, (rows, 1), 0)
    return (gid < N_REAL).astype(jnp.float32)


def _elu(x):
    return jnp.where(x > 0, x, jnp.exp(jnp.minimum(x, 0.0)) - 1.0)


def _gin_pre_stage1(h, agg, w1, b1):
    din = h.shape[1]

    def kern(h_ref, agg_ref, w1_ref, b1_ref, z_ref, st_ref):
        i = pl.program_id(0)
        z = jnp.dot(h_ref[...] + agg_ref[...], w1_ref[...],
                    preferred_element_type=jnp.float32) + b1_ref[...]
        z = z * _row_valid(i, TILE)
        z_ref[...] = z

        @pl.when(i == 0)
        def _():
            st_ref[...] = jnp.zeros_like(st_ref)

        s1 = jnp.sum(z, axis=0, keepdims=True)
        s2 = jnp.sum(z * z, axis=0, keepdims=True)
        upd = jnp.concatenate(
            [s1, s2, jnp.zeros((6, z.shape[1]), jnp.float32)], axis=0)
        st_ref[...] = st_ref[...] + upd

    grid = NPAD // TILE
    return pl.pallas_call(
        kern,
        grid=(grid,),
        in_specs=[
            pl.BlockSpec((TILE, din), lambda i: (i, 0)),
            pl.BlockSpec((TILE, din), lambda i: (i, 0)),
            pl.BlockSpec((din, 512), lambda i: (0, 0)),
            pl.BlockSpec((1, 512), lambda i: (0, 0)),
        ],
        out_specs=[
            pl.BlockSpec((TILE, 512), lambda i: (i, 0)),
            pl.BlockSpec((8, 512), lambda i: (0, 0)),
        ],
        out_shape=[
            jax.ShapeDtypeStruct((NPAD, 512), jnp.float32),
            jax.ShapeDtypeStruct((8, 512), jnp.float32),
        ],
    )(h, agg, w1, b1.reshape(1, 512))


def _gin_pre_stage2(z1, stats, gamma, beta, w2, b2):
    def kern(z_ref, st_ref, g_ref, be_ref, w2_ref, b2_ref, o_ref):
        i = pl.program_id(0)
        st = st_ref[...]
        mu = st[0:1, :] / N_REAL
        var = st[1:2, :] / N_REAL - mu * mu
        zn = g_ref[...] * (z_ref[...] - mu) * lax.rsqrt(var + 1e-5) + be_ref[...]
        a = _elu(zn)
        h = _elu(jnp.dot(a, w2_ref[...],
                         preferred_element_type=jnp.float32) + b2_ref[...])
        o_ref[...] = h * _row_valid(i, TILE)

    grid = NPAD // TILE
    return pl.pallas_call(
        kern,
        grid=(grid,),
        in_specs=[
            pl.BlockSpec((TILE, 512), lambda i: (i, 0)),
            pl.BlockSpec((8, 512), lambda i: (0, 0)),
            pl.BlockSpec((1, 512), lambda i: (0, 0)),
            pl.BlockSpec((1, 512), lambda i: (0, 0)),
            pl.BlockSpec((512, 512), lambda i: (0, 0)),
            pl.BlockSpec((1, 512), lambda i: (0, 0)),
        ],
        out_specs=pl.BlockSpec((TILE, 512), lambda i: (i, 0)),
        out_shape=jax.ShapeDtypeStruct((NPAD, 512), jnp.float32),
    )(z1, stats, gamma.reshape(1, 512), beta.reshape(1, 512),
      w2, b2.reshape(1, 512))


def _pool_score(h, w):
    def kern(h_ref, w_ref, o_ref):
        i = pl.program_id(0)
        wv = w_ref[...]
        nrm = jnp.sqrt(jnp.sum(wv * wv))
        s = jnp.sum(h_ref[...] * wv, axis=1, keepdims=True) / nrm
        s = jnp.tanh(s)
        valid = _row_valid(i, TILE)
        s = s * valid + (valid - 1.0) * 2.0
        o_ref[...] = s

    grid = NPAD // TILE
    return pl.pallas_call(
        kern,
        grid=(grid,),
        in_specs=[
            pl.BlockSpec((TILE, 512), lambda i: (i, 0)),
            pl.BlockSpec((1, 512), lambda i: (0, 0)),
        ],
        out_specs=pl.BlockSpec((TILE, 1), lambda i: (i, 0)),
        out_shape=jax.ShapeDtypeStruct((NPAD, 1), jnp.float32),
    )(h, w.reshape(1, 512))


def _topk_gate(score, k):
    def kern(s_ref, gate_ref, sel_ref):
        s = s_ref[...]
        u = lax.bitcast_convert_type(s, jnp.uint32)
        m = jnp.where(
            (u >> 31) > 0, jnp.uint32(0xFFFFFFFF), jnp.uint32(0x80000000))
        key = u ^ m

        def tbody(j, t):
            bit = (31 - j).astype(jnp.uint32)
            tt = t | (jnp.uint32(1) << bit)
            cnt = jnp.sum((key >= tt).astype(jnp.int32))
            return jnp.where(cnt >= k, tt, t)

        thr = lax.fori_loop(0, 32, tbody, jnp.uint32(0))
        gt = key > thr
        eq = key == thr
        cnt_gt = jnp.sum(gt.astype(jnp.int32))
        need = k - cnt_gt
        ridx = (lax.broadcasted_iota(jnp.int32, s.shape, 0) * 128
                + lax.broadcasted_iota(jnp.int32, s.shape, 1))

        def ibody(j, iv):
            bit = 13 - j
            it = iv + (jnp.int32(1) << bit)
            cnt = jnp.sum((eq & (ridx < it)).astype(jnp.int32))
            return jnp.where(cnt <= need, it, iv)

        idx_thr = lax.fori_loop(0, 14, ibody, jnp.int32(0))
        sel = gt | (eq & (ridx < idx_thr))
        gate_ref[...] = jnp.where(sel, s, 0.0)
        sel_ref[...] = sel.astype(jnp.float32)

    return pl.pallas_call(
        kern,
        out_shape=[
            jax.ShapeDtypeStruct(score.shape, jnp.float32),
            jax.ShapeDtypeStruct(score.shape, jnp.float32),
        ],
    )(score)


def _apply_gate(h, gate):
    def kern(h_ref, g_ref, o_ref):
        o_ref[...] = h_ref[...] * g_ref[...]

    grid = NPAD // TILE
    return pl.pallas_call(
        kern,
        grid=(grid,),
        in_specs=[
            pl.BlockSpec((TILE, 512), lambda i: (i, 0)),
            pl.BlockSpec((TILE, 1), lambda i: (i, 0)),
        ],
        out_specs=pl.BlockSpec((TILE, 512), lambda i: (i, 0)),
        out_shape=jax.ShapeDtypeStruct((NPAD, 512), jnp.float32),
    )(h, gate)


def _gin_post(hp, agg, w1, b1, w2, b2, sel):
    def kern(h_ref, a_ref, w1_ref, b1_ref, w2_ref, b2_ref, s_ref, o_ref):
        z = h_ref[...] + a_ref[...]
        a = _elu(jnp.dot(z, w1_ref[...],
                         preferred_element_type=jnp.float32) + b1_ref[...])
        h = _elu(jnp.dot(a, w2_ref[...],
                         preferred_element_type=jnp.float32) + b2_ref[...])
        o_ref[...] = h * s_ref[...]

    grid = NPAD // TILE
    return pl.pallas_call(
        kern,
        grid=(grid,),
        in_specs=[
            pl.BlockSpec((TILE, 512), lambda i: (i, 0)),
            pl.BlockSpec((TILE, 512), lambda i: (i, 0)),
            pl.BlockSpec((512, 512), lambda i: (0, 0)),
            pl.BlockSpec((1, 512), lambda i: (0, 0)),
            pl.BlockSpec((512, 512), lambda i: (0, 0)),
            pl.BlockSpec((1, 512), lambda i: (0, 0)),
            pl.BlockSpec((TILE, 1), lambda i: (i, 0)),
        ],
        out_specs=pl.BlockSpec((TILE, 512), lambda i: (i, 0)),
        out_shape=jax.ShapeDtypeStruct((NPAD, 512), jnp.float32),
    )(hp, agg, w1, b1.reshape(1, 512), w2, b2.reshape(1, 512), sel)


def _pool_head(h, hw1, hb1, hw2, hb2, hw3p, hb3p):
    def kern(h_ref, w1_ref, b1_ref, w2_ref, b2_ref, w3_ref, b3_ref,
             o_ref, acc_ref):
        i = pl.program_id(0)

        @pl.when(i == 0)
        def _():
            acc_ref[...] = jnp.zeros_like(acc_ref)

        hb = h_ref[...]
        s1 = jnp.sum(hb, axis=0, keepdims=True)
        acc_ref[...] = acc_ref[...] + jnp.concatenate(
            [s1, jnp.zeros((7, 512), jnp.float32)], axis=0)

        @pl.when(i == pl.num_programs(0) - 1)
        def _():
            g = acc_ref[0:1, :]
            g = _elu(jnp.dot(g, w1_ref[...],
                             preferred_element_type=jnp.float32) + b1_ref[...])
            g = _elu(jnp.dot(g, w2_ref[...],
                             preferred_element_type=jnp.float32) + b2_ref[...])
            z = jnp.dot(g, w3_ref[...],
                        preferred_element_type=jnp.float32) + b3_ref[...]
            mx = jnp.max(z, axis=1, keepdims=True)
            lse = jnp.log(jnp.sum(jnp.exp(z - mx), axis=1, keepdims=True))
            out = z - mx - lse
            o_ref[...] = jnp.broadcast_to(out, (8, 128))

    grid = NPAD // TILE
    return pl.pallas_call(
        kern,
        grid=(grid,),
        in_specs=[
            pl.BlockSpec((TILE, 512), lambda i: (i, 0)),
            pl.BlockSpec((512, 512), lambda i: (0, 0)),
            pl.BlockSpec((1, 512), lambda i: (0, 0)),
            pl.BlockSpec((512, 256), lambda i: (0, 0)),
            pl.BlockSpec((1, 256), lambda i: (0, 0)),
            pl.BlockSpec((256, 128), lambda i: (0, 0)),
            pl.BlockSpec((1, 128), lambda i: (0, 0)),
        ],
        out_specs=pl.BlockSpec((8, 128), lambda i: (0, 0)),
        out_shape=jax.ShapeDtypeStruct((8, 128), jnp.float32),
        scratch_shapes=[pltpu.VMEM((8, 512), jnp.float32)],
    )(h, hw1, hb1.reshape(1, 512), hw2, hb2.reshape(1, 256), hw3p, hb3p)


@jax.jit
def _run(x, edge_index, params):
    n = x.shape[0]
    k = int(ceil(0.1 * n))

    h = jnp.zeros((NPAD, x.shape[1]), jnp.float32).at[:n].set(x)
    src = edge_index[0].astype(jnp.int32)
    dst = edge_index[1].astype(jnp.int32)
    pad_e = EPAD - src.shape[0]
    src3 = jnp.concatenate(
        [src, jnp.zeros((pad_e,), jnp.int32)]).reshape(NSUB, NBLK, EBLK)
    dst3 = jnp.concatenate(
        [dst, jnp.full((pad_e,), DUMMY_DST, jnp.int32)]
    ).reshape(NSUB, NBLK, EBLK)
    zeros_chunk = jnp.zeros((NPAD, CCHUNK), jnp.float32)

    for p in params["pre"]:
        agg = _segment_sum(h, src3, dst3, zeros_chunk)
        z1, stats = _gin_pre_stage1(h, agg, p["W1"], p["b1"])
        h = _gin_pre_stage2(z1, stats, p["gamma"], p["beta"], p["W2"], p["b2"])

    score = _pool_score(h, params["pool_w"])
    gate2d, sel2d = _topk_gate(score.reshape(NPAD // 128, 128), k)
    gate = gate2d.reshape(NPAD, 1)
    sel = sel2d.reshape(NPAD, 1)
    h = _apply_gate(h, gate)

    for p in params["post"]:
        agg = _segment_sum(h, src3, dst3, zeros_chunk)
        h = _gin_post(h, agg, p["W1"], p["b1"], p["W2"], p["b2"], sel)

    hd = params["head"]
    w3p = jnp.zeros((256, 128), jnp.float32).at[:, :10].set(hd["W3"])
    b3p = jnp.full((1, 128), -1e30, jnp.float32).at[0, :10].set(hd["b3"])
    res = _pool_head(h, hd["W1"], hd["b1"], hd["W2"], hd["b2"], w3p, b3p)
    return res[0:1, 0:10]


def kernel(x, edge_index, batch, params):
    out = _run(x, edge_index, params)
    return (out, jnp.float32(0.0))


# speedup vs baseline: 1.7639x; 1.1442x over previous
import functools
from math import ceil

import jax
import jax.numpy as jnp
from jax import lax
from jax.experimental import pallas as pl
from jax.experimental.pallas import tpu as pltpu
from jax.experimental.pallas import tpu_sc as plsc

N_REAL = 10000
NPAD = 10240
E_REAL = 160000
EPAD = 163840
NSUB = 16
NCORE = 1
NW = NSUB * NCORE
EPW = EPAD // NW
EBLK = 128
NBUF = 2
NBLK = EPW // EBLK
ROWS_PW = NPAD // NSUB
CCHUNK = 128
TILE = 1024
DUMMY_DST = N_REAL + 8


@functools.cache
def _make_segsum_sc():
    @functools.partial(
        pl.kernel,
        mesh=plsc.VectorSubcoreMesh(
            core_axis_name="c", subcore_axis_name="s", num_cores=NCORE),
        out_type=jax.ShapeDtypeStruct((NPAD, CCHUNK), jnp.float32),
        scratch_types=(
            [pltpu.VMEM((NBUF, EBLK), jnp.int32)] * 2
            + [pltpu.VMEM((EBLK, CCHUNK), jnp.float32)] * NBUF
            + [pltpu.VMEM_SHARED((NPAD, CCHUNK), jnp.float32)]
            + [pltpu.SemaphoreType.DMA] * NBUF
        ),
    )
    def _segsum_sc(h_hbm, src_hbm, dst_hbm, zeros_hbm, out_hbm,
                   sidx, didx, r0, r1, acc, s0, s1):
        sid = lax.axis_index("s")
        wid = sid
        rows = [r0, r1]
        sems = [s0, s1]
        base = sid * ROWS_PW
        pltpu.sync_copy(zeros_hbm.at[pl.ds(base, ROWS_PW)],
                        acc.at[pl.ds(base, ROWS_PW)])
        plsc.subcore_barrier()

        def body(j, carry):
            pltpu.sync_copy(src_hbm.at[wid, pl.ds(j * NBUF, NBUF)], sidx)
            pltpu.sync_copy(dst_hbm.at[wid, pl.ds(j * NBUF, NBUF)], didx)
            cps = [pltpu.async_copy(h_hbm.at[sidx.at[b]], rows[b], sems[b])
                   for b in range(NBUF)]
            for b in range(NBUF):
                cps[b].wait()
                pltpu.sync_copy(rows[b], acc.at[didx.at[b]], add=True)
            return carry

        lax.fori_loop(0, NBLK // NBUF, body, 0)
        plsc.subcore_barrier()
        pltpu.sync_copy(acc.at[pl.ds(base, ROWS_PW)],
                        out_hbm.at[pl.ds(base, ROWS_PW)])

    return _segsum_sc


def _segment_sum(h, src3, dst3, zeros_chunk):
    d = h.shape[1]
    fn = _make_segsum_sc()
    outs = []
    for c in range(d // CCHUNK):
        hc = lax.slice_in_dim(h, c * CCHUNK, (c + 1) * CCHUNK, axis=1)
        outs.append(fn(hc, src3, dst3, zeros_chunk))
    return jnp.concatenate(outs, axis=1)


def _row_valid(i, rows):
    gid = i * TILE + lax.broadcasted_iota(jnp.int32, (rows, 1), 0)
    return (gid < N_REAL).astype(jnp.float32)


def _elu(x):
    return jnp.where(x > 0, x, jnp.exp(jnp.minimum(x, 0.0)) - 1.0)


def _gin_pre_stage1(h, agg, w1, b1):
    din = h.shape[1]

    def kern(h_ref, agg_ref, w1_ref, b1_ref, z_ref, st_ref):
        i = pl.program_id(0)
        z = jnp.dot(h_ref[...] + agg_ref[...], w1_ref[...],
                    preferred_element_type=jnp.float32) + b1_ref[...]
        z = z * _row_valid(i, TILE)
        z_ref[...] = z

        @pl.when(i == 0)
        def _():
            st_ref[...] = jnp.zeros_like(st_ref)

        s1 = jnp.sum(z, axis=0, keepdims=True)
        s2 = jnp.sum(z * z, axis=0, keepdims=True)
        upd = jnp.concatenate(
            [s1, s2, jnp.zeros((6, z.shape[1]), jnp.float32)], axis=0)
        st_ref[...] = st_ref[...] + upd

    grid = NPAD // TILE
    return pl.pallas_call(
        kern,
        grid=(grid,),
        in_specs=[
            pl.BlockSpec((TILE, din), lambda i: (i, 0)),
            pl.BlockSpec((TILE, din), lambda i: (i, 0)),
            pl.BlockSpec((din, 512), lambda i: (0, 0)),
            pl.BlockSpec((1, 512), lambda i: (0, 0)),
        ],
        out_specs=[
            pl.BlockSpec((TILE, 512), lambda i: (i, 0)),
            pl.BlockSpec((8, 512), lambda i: (0, 0)),
        ],
        out_shape=[
            jax.ShapeDtypeStruct((NPAD, 512), jnp.float32),
            jax.ShapeDtypeStruct((8, 512), jnp.float32),
        ],
    )(h, agg, w1, b1.reshape(1, 512))


def _gin_pre_stage2(z1, stats, gamma, beta, w2, b2):
    def kern(z_ref, st_ref, g_ref, be_ref, w2_ref, b2_ref, o_ref):
        i = pl.program_id(0)
        st = st_ref[...]
        mu = st[0:1, :] / N_REAL
        var = st[1:2, :] / N_REAL - mu * mu
        zn = g_ref[...] * (z_ref[...] - mu) * lax.rsqrt(var + 1e-5) + be_ref[...]
        a = _elu(zn)
        h = _elu(jnp.dot(a, w2_ref[...],
                         preferred_element_type=jnp.float32) + b2_ref[...])
        o_ref[...] = h * _row_valid(i, TILE)

    grid = NPAD // TILE
    return pl.pallas_call(
        kern,
        grid=(grid,),
        in_specs=[
            pl.BlockSpec((TILE, 512), lambda i: (i, 0)),
            pl.BlockSpec((8, 512), lambda i: (0, 0)),
            pl.BlockSpec((1, 512), lambda i: (0, 0)),
            pl.BlockSpec((1, 512), lambda i: (0, 0)),
            pl.BlockSpec((512, 512), lambda i: (0, 0)),
            pl.BlockSpec((1, 512), lambda i: (0, 0)),
        ],
        out_specs=pl.BlockSpec((TILE, 512), lambda i: (i, 0)),
        out_shape=jax.ShapeDtypeStruct((NPAD, 512), jnp.float32),
    )(z1, stats, gamma.reshape(1, 512), beta.reshape(1, 512),
      w2, b2.reshape(1, 512))


def _pool_score(h, w):
    def kern(h_ref, w_ref, o_ref):
        i = pl.program_id(0)
        wv = w_ref[...]
        nrm = jnp.sqrt(jnp.sum(wv * wv))
        s = jnp.sum(h_ref[...] * wv, axis=1, keepdims=True) / nrm
        s = jnp.tanh(s)
        valid = _row_valid(i, TILE)
        s = s * valid + (valid - 1.0) * 2.0
        o_ref[...] = s

    grid = NPAD // TILE
    return pl.pallas_call(
        kern,
        grid=(grid,),
        in_specs=[
            pl.BlockSpec((TILE, 512), lambda i: (i, 0)),
            pl.BlockSpec((1, 512), lambda i: (0, 0)),
        ],
        out_specs=pl.BlockSpec((TILE, 1), lambda i: (i, 0)),
        out_shape=jax.ShapeDtypeStruct((NPAD, 1), jnp.float32),
    )(h, w.reshape(1, 512))


def _topk_gate(score, k):
    def kern(s_ref, gate_ref, sel_ref):
        s = s_ref[...]
        u = lax.bitcast_convert_type(s, jnp.uint32)
        m = jnp.where(
            (u >> 31) > 0, jnp.uint32(0xFFFFFFFF), jnp.uint32(0x80000000))
        key = u ^ m

        def tbody(j, t):
            bit = (31 - j).astype(jnp.uint32)
            tt = t | (jnp.uint32(1) << bit)
            cnt = jnp.sum((key >= tt).astype(jnp.int32))
            return jnp.where(cnt >= k, tt, t)

        thr = lax.fori_loop(0, 32, tbody, jnp.uint32(0))
        gt = key > thr
        eq = key == thr
        cnt_gt = jnp.sum(gt.astype(jnp.int32))
        need = k - cnt_gt
        ridx = (lax.broadcasted_iota(jnp.int32, s.shape, 0) * 128
                + lax.broadcasted_iota(jnp.int32, s.shape, 1))

        def ibody(j, iv):
            bit = 13 - j
            it = iv + (jnp.int32(1) << bit)
            cnt = jnp.sum((eq & (ridx < it)).astype(jnp.int32))
            return jnp.where(cnt <= need, it, iv)

        idx_thr = lax.fori_loop(0, 14, ibody, jnp.int32(0))
        sel = gt | (eq & (ridx < idx_thr))
        gate_ref[...] = jnp.where(sel, s, 0.0)
        sel_ref[...] = sel.astype(jnp.float32)

    return pl.pallas_call(
        kern,
        out_shape=[
            jax.ShapeDtypeStruct(score.shape, jnp.float32),
            jax.ShapeDtypeStruct(score.shape, jnp.float32),
        ],
    )(score)


def _apply_gate(h, gate):
    def kern(h_ref, g_ref, o_ref):
        o_ref[...] = h_ref[...] * g_ref[...]

    grid = NPAD // TILE
    return pl.pallas_call(
        kern,
        grid=(grid,),
        in_specs=[
            pl.BlockSpec((TILE, 512), lambda i: (i, 0)),
            pl.BlockSpec((TILE, 1), lambda i: (i, 0)),
        ],
        out_specs=pl.BlockSpec((TILE, 512), lambda i: (i, 0)),
        out_shape=jax.ShapeDtypeStruct((NPAD, 512), jnp.float32),
    )(h, gate)


def _gin_post(hp, agg, w1, b1, w2, b2, sel):
    def kern(h_ref, a_ref, w1_ref, b1_ref, w2_ref, b2_ref, s_ref, o_ref):
        z = h_ref[...] + a_ref[...]
        a = _elu(jnp.dot(z, w1_ref[...],
                         preferred_element_type=jnp.float32) + b1_ref[...])
        h = _elu(jnp.dot(a, w2_ref[...],
                         preferred_element_type=jnp.float32) + b2_ref[...])
        o_ref[...] = h * s_ref[...]

    grid = NPAD // TILE
    return pl.pallas_call(
        kern,
        grid=(grid,),
        in_specs=[
            pl.BlockSpec((TILE, 512), lambda i: (i, 0)),
            pl.BlockSpec((TILE, 512), lambda i: (i, 0)),
            pl.BlockSpec((512, 512), lambda i: (0, 0)),
            pl.BlockSpec((1, 512), lambda i: (0, 0)),
            pl.BlockSpec((512, 512), lambda i: (0, 0)),
            pl.BlockSpec((1, 512), lambda i: (0, 0)),
            pl.BlockSpec((TILE, 1), lambda i: (i, 0)),
        ],
        out_specs=pl.BlockSpec((TILE, 512), lambda i: (i, 0)),
        out_shape=jax.ShapeDtypeStruct((NPAD, 512), jnp.float32),
    )(hp, agg, w1, b1.reshape(1, 512), w2, b2.reshape(1, 512), sel)


def _pool_head(h, hw1, hb1, hw2, hb2, hw3p, hb3p):
    def kern(h_ref, w1_ref, b1_ref, w2_ref, b2_ref, w3_ref, b3_ref,
             o_ref, acc_ref):
        i = pl.program_id(0)

        @pl.when(i == 0)
        def _():
            acc_ref[...] = jnp.zeros_like(acc_ref)

        hb = h_ref[...]
        s1 = jnp.sum(hb, axis=0, keepdims=True)
        acc_ref[...] = acc_ref[...] + jnp.concatenate(
            [s1, jnp.zeros((7, 512), jnp.float32)], axis=0)

        @pl.when(i == pl.num_programs(0) - 1)
        def _():
            g = acc_ref[0:1, :]
            g = _elu(jnp.dot(g, w1_ref[...],
                             preferred_element_type=jnp.float32) + b1_ref[...])
            g = _elu(jnp.dot(g, w2_ref[...],
                             preferred_element_type=jnp.float32) + b2_ref[...])
            z = jnp.dot(g, w3_ref[...],
                        preferred_element_type=jnp.float32) + b3_ref[...]
            mx = jnp.max(z, axis=1, keepdims=True)
            lse = jnp.log(jnp.sum(jnp.exp(z - mx), axis=1, keepdims=True))
            out = z - mx - lse
            o_ref[...] = jnp.broadcast_to(out, (8, 128))

    grid = NPAD // TILE
    return pl.pallas_call(
        kern,
        grid=(grid,),
        in_specs=[
            pl.BlockSpec((TILE, 512), lambda i: (i, 0)),
            pl.BlockSpec((512, 512), lambda i: (0, 0)),
            pl.BlockSpec((1, 512), lambda i: (0, 0)),
            pl.BlockSpec((512, 256), lambda i: (0, 0)),
            pl.BlockSpec((1, 256), lambda i: (0, 0)),
            pl.BlockSpec((256, 128), lambda i: (0, 0)),
            pl.BlockSpec((1, 128), lambda i: (0, 0)),
        ],
        out_specs=pl.BlockSpec((8, 128), lambda i: (0, 0)),
        out_shape=jax.ShapeDtypeStruct((8, 128), jnp.float32),
        scratch_shapes=[pltpu.VMEM((8, 512), jnp.float32)],
    )(h, hw1, hb1.reshape(1, 512), hw2, hb2.reshape(1, 256), hw3p, hb3p)


@jax.jit
def _run(x, edge_index, params):
    n = x.shape[0]
    k = int(ceil(0.1 * n))

    h = jnp.zeros((NPAD, x.shape[1]), jnp.float32).at[:n].set(x)
    src = edge_index[0].astype(jnp.int32)
    dst = edge_index[1].astype(jnp.int32)
    pad_e = EPAD - src.shape[0]
    src3 = jnp.concatenate(
        [src, jnp.zeros((pad_e,), jnp.int32)]).reshape(NW, NBLK, EBLK)
    dst3 = jnp.concatenate(
        [dst, jnp.full((pad_e,), DUMMY_DST, jnp.int32)]
    ).reshape(NW, NBLK, EBLK)
    zeros_chunk = jnp.zeros((NPAD, CCHUNK), jnp.float32)

    for p in params["pre"]:
        agg = _segment_sum(h, src3, dst3, zeros_chunk)
        z1, stats = _gin_pre_stage1(h, agg, p["W1"], p["b1"])
        h = _gin_pre_stage2(z1, stats, p["gamma"], p["beta"], p["W2"], p["b2"])

    score = _pool_score(h, params["pool_w"])
    gate2d, sel2d = _topk_gate(score.reshape(NPAD // 128, 128), k)
    gate = gate2d.reshape(NPAD, 1)
    sel = sel2d.reshape(NPAD, 1)
    h = _apply_gate(h, gate)

    for p in params["post"]:
        agg = _segment_sum(h, src3, dst3, zeros_chunk)
        h = _gin_post(h, agg, p["W1"], p["b1"], p["W2"], p["b2"], sel)

    hd = params["head"]
    w3p = jnp.zeros((256, 128), jnp.float32).at[:, :10].set(hd["W3"])
    b3p = jnp.full((1, 128), -1e30, jnp.float32).at[0, :10].set(hd["b3"])
    res = _pool_head(h, hd["W1"], hd["b1"], hd["W2"], hd["b2"], w3p, b3p)
    return res[0:1, 0:10]


def kernel(x, edge_index, batch, params):
    out = _run(x, edge_index, params)
    return (out, jnp.float32(0.0))
